# baseline (speedup 1.0000x reference)
import functools

import jax
import jax.numpy as jnp
from jax.experimental import pallas as pl
from jax.experimental.pallas import tpu as pltpu

_K = 10


def _retrieve_body(qk_ref, mk_ref, mv_ref, mpos_ref, mask_ref,
                   rk_ref, rv_ref, pos_ref, mask_out_ref):
    bq = qk_ref.shape[0]
    m = mk_ref.shape[0]
    s = mask_ref.shape[1]

    qk = qk_ref[...]
    mk = mk_ref[...]
    qn = qk / (jnp.sqrt(jnp.sum(qk * qk, axis=1, keepdims=True)) + 1e-8)
    mn = mk / (jnp.sqrt(jnp.sum(mk * mk, axis=1, keepdims=True)) + 1e-8)
    sims = jax.lax.dot_general(
        qn, mn, (((1,), (1,)), ((), ())), preferred_element_type=jnp.float32)

    iota = jax.lax.broadcasted_iota(jnp.int32, (bq, m), 1)
    mpos = mpos_ref[...]
    mv = mv_ref[...]
    cur = sims
    pos_cols = []
    for j in range(_K):
        mx = jnp.max(cur, axis=1, keepdims=True)
        hit = cur == mx
        sel = jnp.min(jnp.where(hit, iota, m), axis=1, keepdims=True)
        here = iota == sel
        onehot = here.astype(jnp.float32)
        rk_ref[:, j, :] = jax.lax.dot_general(
            onehot, mk, (((1,), (0,)), ((), ())),
            preferred_element_type=jnp.float32)
        rv_ref[:, j, :] = jax.lax.dot_general(
            onehot, mv, (((1,), (0,)), ((), ())),
            preferred_element_type=jnp.float32)
        pos_cols.append(jnp.sum(jnp.where(here, mpos, 0.0), axis=1, keepdims=True))
        cur = jnp.where(here, -jnp.inf, cur)

    mask_out_ref[:, :_K] = jnp.ones((bq, _K), mask_out_ref.dtype)
    mask_out_ref[:, _K:] = mask_ref[...]
    pos_ref[:, :s] = jax.lax.broadcasted_iota(jnp.int32, (bq, s), 1).astype(jnp.float32)
    pos_ref[:, s:] = jnp.concatenate(pos_cols, axis=1)


def _concat_body(rk_ref, rv_ref, k_ref, v_ref, ok_ref, ov_ref):
    ok_ref[0, 0, :_K, :] = rk_ref[0, :, :]
    ov_ref[0, 0, :_K, :] = rv_ref[0, :, :]
    ok_ref[0, 0, _K:, :] = k_ref[0, 0, :, :]
    ov_ref[0, 0, _K:, :] = v_ref[0, 0, :, :]


def kernel(inputs, q, k, v, attention_mask, mem_keys, mem_values,
           mem_positions, seq_len_q):
    b, h, s, dh = q.shape
    m = mem_keys.shape[0]

    query_key = k[:, :, s - 1, :].reshape(b, h * dh)
    mpos2 = mem_positions.reshape(1, m)

    retr_k, retr_v, positions_k, mask_aug = pl.pallas_call(
        _retrieve_body,
        out_shape=(
            jax.ShapeDtypeStruct((b, _K, h * dh), jnp.float32),
            jax.ShapeDtypeStruct((b, _K, h * dh), jnp.float32),
            jax.ShapeDtypeStruct((b, s + _K), jnp.float32),
            jax.ShapeDtypeStruct((b, s + _K), attention_mask.dtype),
        ),
    )(query_key, mem_keys, mem_values, mpos2, attention_mask)

    k_aug, v_aug = pl.pallas_call(
        _concat_body,
        grid=(h, b),
        in_specs=[
            pl.BlockSpec((1, _K, dh), lambda hh, bb: (bb, 0, hh)),
            pl.BlockSpec((1, _K, dh), lambda hh, bb: (bb, 0, hh)),
            pl.BlockSpec((1, 1, s, dh), lambda hh, bb: (bb, hh, 0, 0)),
            pl.BlockSpec((1, 1, s, dh), lambda hh, bb: (bb, hh, 0, 0)),
        ],
        out_specs=[
            pl.BlockSpec((1, 1, _K + s, dh), lambda hh, bb: (bb, hh, 0, 0)),
            pl.BlockSpec((1, 1, _K + s, dh), lambda hh, bb: (bb, hh, 0, 0)),
        ],
        out_shape=[
            jax.ShapeDtypeStruct((b, h, _K + s, dh), jnp.float32),
            jax.ShapeDtypeStruct((b, h, _K + s, dh), jnp.float32),
        ],
    )(retr_k, retr_v, k, v)

    return (inputs, q, k_aug, v_aug, mask_aug, _K + s, positions_k)
